# Initial kernel scaffold; baseline (speedup 1.0000x reference)
import jax
import jax.numpy as jnp
from jax.experimental import pallas as pl


def kernel(x, W0, W1, W2):
    raise NotImplementedError("write your pallas kernel here")


# speedup vs baseline: 2.6355x; 2.6355x over previous
import functools

import jax
import jax.numpy as jnp
from jax.experimental import pallas as pl
from jax.experimental.pallas import tpu as pltpu
from jax.experimental.pallas import tpu_sc as plsc

_NUM_CLUSTERS = 1000
_ROWS = _NUM_CLUSTERS + 1
_EMB = 128
_GW = 128


def _sc_gather(table, idx2d, n):
    mesh = plsc.VectorSubcoreMesh(core_axis_name="c", subcore_axis_name="s")

    @functools.partial(
        pl.kernel,
        out_type=jax.ShapeDtypeStruct((n, _EMB), jnp.float32),
        mesh=mesh,
    )
    def k(table_hbm, idx_hbm, out_hbm):
        def body(i_vmem, o_vmem):
            pltpu.sync_copy(table_hbm.at[i_vmem.at[0]], o_vmem)

        pltpu.emit_pipeline(
            body,
            grid=(n // _GW,),
            in_specs=[pl.BlockSpec((1, _GW), lambda i: (0, i))],
            out_specs=[pl.BlockSpec((_GW, _EMB), lambda i: (i, 0))],
            core_axis_name=("c", "s"),
            dimension_semantics=(pltpu.PARALLEL,),
        )(idx_hbm, out_hbm)

    return k(table, idx2d)


def kernel(x, W0, W1, W2):
    B, T, L = x.shape
    n = B * T * L
    table = jnp.concatenate([W0, W1, W2], axis=0)
    offs = (jnp.arange(L, dtype=jnp.int32) * _ROWS).astype(x.dtype)
    idx = (x + offs).reshape(1, n)
    out = _sc_gather(table, idx, n)
    return out.reshape(B, T, L, _EMB)


# speedup vs baseline: 9.5100x; 3.6085x over previous
import functools

import jax
import jax.numpy as jnp
from jax.experimental import pallas as pl
from jax.experimental.pallas import tpu as pltpu
from jax.experimental.pallas import tpu_sc as plsc

_NUM_CLUSTERS = 1000
_ROWS = _NUM_CLUSTERS + 1
_EMB = 128
_GW = 128


def _sc_gather(table, idx_ltb, B, T, L):
    mesh = plsc.VectorSubcoreMesh(core_axis_name="c", subcore_axis_name="s")
    nb = B // _GW

    @functools.partial(
        pl.kernel,
        out_type=jax.ShapeDtypeStruct((T, L, B, _EMB), jnp.float32),
        mesh=mesh,
    )
    def k(table_hbm, idx_hbm, out_hbm):
        def body(i_vmem, o_vmem):
            pltpu.sync_copy(table_hbm.at[i_vmem.at[0, 0]], o_vmem.at[0, 0])

        pltpu.emit_pipeline(
            body,
            grid=(T * L * nb,),
            in_specs=[
                pl.BlockSpec(
                    (1, 1, _GW),
                    index_map=lambda i: ((i // nb) % L, i // (L * nb), i % nb),
                )
            ],
            out_specs=[
                pl.BlockSpec(
                    (1, 1, _GW, _EMB),
                    index_map=lambda i: (i // (L * nb), (i // nb) % L, i % nb, 0),
                )
            ],
            core_axis_name=("c", "s"),
            dimension_semantics=(pltpu.PARALLEL,),
        )(idx_hbm, out_hbm)

    return k(table, idx_ltb)


def kernel(x, W0, W1, W2):
    B, T, L = x.shape
    table = jnp.concatenate([W0, W1, W2], axis=0)
    offs = (jnp.arange(L, dtype=jnp.int32) * _ROWS).astype(x.dtype)
    idx_ltb = jnp.transpose(x + offs, (2, 1, 0))
    out = _sc_gather(table, idx_ltb, B, T, L)
    return jnp.transpose(out, (2, 0, 1, 3))


# speedup vs baseline: 19.3016x; 2.0296x over previous
import functools

import jax
import jax.numpy as jnp
from jax.experimental import pallas as pl
from jax.experimental.pallas import tpu as pltpu
from jax.experimental.pallas import tpu_sc as plsc

_NUM_CLUSTERS = 1000
_ROWS = _NUM_CLUSTERS + 1
_EMB = 128
_GW = 128


def _sc_gather(table, idx_ltb, B, T, L):
    mesh = plsc.VectorSubcoreMesh(core_axis_name="c", subcore_axis_name="s")
    nb = B // _GW

    @functools.partial(
        pl.kernel,
        out_type=jax.ShapeDtypeStruct((T, L, B, _EMB), jnp.float32),
        mesh=mesh,
        scratch_types=[pltpu.VMEM_SHARED((L * _ROWS, _EMB), jnp.float32)],
    )
    def k(table_hbm, idx_hbm, out_hbm, table_sh):
        sid = jax.lax.axis_index("s")

        @pl.when(sid == 0)
        def _():
            pltpu.sync_copy(table_hbm, table_sh)

        plsc.subcore_barrier()

        def body(i_vmem, o_vmem):
            pltpu.sync_copy(table_sh.at[i_vmem.at[0, 0]], o_vmem.at[0, 0])

        pltpu.emit_pipeline(
            body,
            grid=(T * L * nb,),
            in_specs=[
                pl.BlockSpec(
                    (1, 1, _GW),
                    index_map=lambda i: ((i // nb) % L, i // (L * nb), i % nb),
                )
            ],
            out_specs=[
                pl.BlockSpec(
                    (1, 1, _GW, _EMB),
                    index_map=lambda i: (i // (L * nb), (i // nb) % L, i % nb, 0),
                )
            ],
            core_axis_name=("c", "s"),
            dimension_semantics=(pltpu.PARALLEL,),
        )(idx_hbm, out_hbm)

    return k(table, idx_ltb)


def kernel(x, W0, W1, W2):
    B, T, L = x.shape
    table = jnp.concatenate([W0, W1, W2], axis=0)
    offs = (jnp.arange(L, dtype=jnp.int32) * _ROWS).astype(x.dtype)
    idx_ltb = jnp.transpose(x + offs, (2, 1, 0))
    out = _sc_gather(table, idx_ltb, B, T, L)
    return jnp.transpose(out, (2, 0, 1, 3))
